# baseline (speedup 1.0000x reference)
import functools

import jax
import jax.numpy as jnp
from jax import lax
from jax.experimental import pallas as pl
from jax.experimental.pallas import tpu as pltpu
from jax.experimental.pallas import tpu_sc as plsc

_NUM_CORES = 1
_NUM_SUBCORES = 16
_LANES = 16
_NW = _NUM_CORES * _NUM_SUBCORES


def _lut_body(vocab, k_per_row, emb_ref, w_ref, b_ref, lut_ref):
    n_combo = vocab ** k_per_row
    r = lax.broadcasted_iota(jnp.int32, (n_combo, vocab), 0)
    v = lax.broadcasted_iota(jnp.int32, (n_combo, vocab), 1)
    counts = jnp.zeros((n_combo, vocab), jnp.float32)
    for slot in range(k_per_row):
        digit = (r // (vocab ** (k_per_row - 1 - slot))) % vocab
        counts = counts + (digit == v).astype(jnp.float32)
    counts = counts * (1.0 / k_per_row)
    pooled = jnp.dot(counts, emb_ref[...], preferred_element_type=jnp.float32)
    logits = jnp.dot(pooled, w_ref[...], preferred_element_type=jnp.float32)
    logits = logits + b_ref[...]
    m = jnp.max(logits, axis=-1, keepdims=True)
    e = jnp.exp(logits - m)
    lut_ref[...] = e / jnp.sum(e, axis=-1, keepdims=True)


def kernel(inputs, emb_table, W, b):
    batch, k_per_row = inputs.shape
    vocab = emb_table.shape[0]
    out_units = W.shape[1]
    n_combo = vocab ** k_per_row
    n_lut = n_combo * out_units

    lut = pl.pallas_call(
        functools.partial(_lut_body, vocab, k_per_row),
        out_shape=jax.ShapeDtypeStruct((n_combo, out_units), jnp.float32),
    )(emb_table, W, b.reshape(1, out_units))

    idx2d = inputs.astype(jnp.int32)
    lut_flat = lut.reshape(n_lut)
    rows_per_w = batch // _NW
    chunk = 256
    n_chunks = rows_per_w // chunk
    iters = chunk // _LANES

    mesh = plsc.VectorSubcoreMesh(
        core_axis_name="c", subcore_axis_name="s",
        num_cores=_NUM_CORES, num_subcores=_NUM_SUBCORES)

    @functools.partial(
        pl.kernel,
        out_type=jax.ShapeDtypeStruct((batch, out_units), jnp.float32),
        mesh=mesh,
        compiler_params=pltpu.CompilerParams(
            needs_layout_passes=False, use_tc_tiling_on_sc=True),
        scratch_types=[
            pltpu.VMEM((chunk, k_per_row), jnp.int32),
            pltpu.VMEM((n_lut,), jnp.float32),
            pltpu.VMEM((chunk, out_units), jnp.float32),
            pltpu.SemaphoreType.DMA,
        ],
    )
    def sc_lookup(idx_hbm, lut_hbm, out_hbm, idx_v, lut_v, out_v, sem):
        wid = lax.axis_index("s") * _NUM_CORES + lax.axis_index("c")
        row0 = wid * rows_per_w
        pltpu.sync_copy(lut_hbm, lut_v)
        lane = lax.iota(jnp.int32, _LANES)
        cols = [jnp.full((_LANES,), s, jnp.int32) for s in range(k_per_row)]
        kcols = [jnp.full((_LANES,), k, jnp.int32) for k in range(out_units)]

        for c in range(n_chunks):
            crow = row0 + c * chunk
            pltpu.sync_copy(idx_hbm.at[pl.ds(crow, chunk), :], idx_v)

            @plsc.parallel_loop(0, iters, unroll=8)
            def body(j):
                rows = j * _LANES + lane
                i0 = plsc.load_gather(idx_v, [rows, cols[0]])
                i1 = plsc.load_gather(idx_v, [rows, cols[1]])
                i2 = plsc.load_gather(idx_v, [rows, cols[2]])
                c3 = (i0 * (vocab * vocab) + i1 * vocab + i2) * out_units
                for k in range(out_units):
                    vals = plsc.load_gather(lut_v, [c3 + k])
                    plsc.store_scatter(out_v, [rows, kcols[k]], vals)

            pltpu.sync_copy(out_v, out_hbm.at[pl.ds(crow, chunk), :])

    return sc_lookup(idx2d, lut_flat)


# speedup vs baseline: 1.1156x; 1.1156x over previous
import functools

import jax
import jax.numpy as jnp
from jax import lax
from jax.experimental import pallas as pl
from jax.experimental.pallas import tpu as pltpu
from jax.experimental.pallas import tpu_sc as plsc

_NUM_CORES = 1
_NUM_SUBCORES = 16
_LANES = 16
_NW = _NUM_CORES * _NUM_SUBCORES


def _lut_body(vocab, k_per_row, emb_ref, w_ref, b_ref, lut_ref):
    n_combo = vocab ** k_per_row
    r = lax.broadcasted_iota(jnp.int32, (n_combo, vocab), 0)
    v = lax.broadcasted_iota(jnp.int32, (n_combo, vocab), 1)
    counts = jnp.zeros((n_combo, vocab), jnp.float32)
    for slot in range(k_per_row):
        digit = (r // (vocab ** (k_per_row - 1 - slot))) % vocab
        counts = counts + (digit == v).astype(jnp.float32)
    counts = counts * (1.0 / k_per_row)
    pooled = jnp.dot(counts, emb_ref[...], preferred_element_type=jnp.float32)
    logits = jnp.dot(pooled, w_ref[...], preferred_element_type=jnp.float32)
    logits = logits + b_ref[...]
    m = jnp.max(logits, axis=-1, keepdims=True)
    e = jnp.exp(logits - m)
    lut_ref[...] = e / jnp.sum(e, axis=-1, keepdims=True)


def kernel(inputs, emb_table, W, b):
    batch, k_per_row = inputs.shape
    vocab = emb_table.shape[0]
    out_units = W.shape[1]
    n_combo = vocab ** k_per_row
    n_lut = n_combo * out_units

    lut = pl.pallas_call(
        functools.partial(_lut_body, vocab, k_per_row),
        out_shape=jax.ShapeDtypeStruct((n_combo, out_units), jnp.float32),
    )(emb_table, W, b.reshape(1, out_units))

    idx2d = inputs.astype(jnp.int32)
    lut_flat = lut.reshape(n_lut)
    rows_per_w = batch // _NW
    chunk = 128
    n_chunks = rows_per_w // chunk
    iters = chunk // _LANES

    mesh = plsc.VectorSubcoreMesh(
        core_axis_name="c", subcore_axis_name="s",
        num_cores=_NUM_CORES, num_subcores=_NUM_SUBCORES)

    @functools.partial(
        pl.kernel,
        out_type=jax.ShapeDtypeStruct((batch, out_units), jnp.float32),
        mesh=mesh,
        compiler_params=pltpu.CompilerParams(needs_layout_passes=False),
        scratch_types=[
            pltpu.VMEM((chunk, k_per_row), jnp.int32),
            pltpu.VMEM((chunk, k_per_row), jnp.int32),
            pltpu.VMEM((n_lut,), jnp.float32),
            pltpu.VMEM((chunk, out_units), jnp.float32),
            pltpu.VMEM((chunk, out_units), jnp.float32),
            pltpu.SemaphoreType.DMA,
            pltpu.SemaphoreType.DMA,
            pltpu.SemaphoreType.DMA,
            pltpu.SemaphoreType.DMA,
        ],
    )
    def sc_lookup(idx_hbm, lut_hbm, out_hbm,
                  idx_a, idx_b, lut_v, out_a, out_b,
                  sem_ia, sem_ib, sem_oa, sem_ob):
        wid = lax.axis_index("s") * _NUM_CORES + lax.axis_index("c")
        row0 = wid * rows_per_w
        idx_bufs, out_bufs = [idx_a, idx_b], [out_a, out_b]
        in_sems, out_sems = [sem_ia, sem_ib], [sem_oa, sem_ob]
        lane = lax.iota(jnp.int32, _LANES)
        cols = [jnp.full((_LANES,), s, jnp.int32) for s in range(k_per_row)]
        kcols = [jnp.full((_LANES,), k, jnp.int32) for k in range(out_units)]

        def compute(idx_v, out_v):
            @plsc.parallel_loop(0, iters, unroll=8)
            def body(j):
                rows = j * _LANES + lane
                i0 = plsc.load_gather(idx_v, [rows, cols[0]])
                i1 = plsc.load_gather(idx_v, [rows, cols[1]])
                i2 = plsc.load_gather(idx_v, [rows, cols[2]])
                c3 = (i0 * (vocab * vocab) + i1 * vocab + i2) * out_units
                for k in range(out_units):
                    vals = plsc.load_gather(lut_v, [c3 + k])
                    plsc.store_scatter(out_v, [rows, kcols[k]], vals)

        in_cp = [None] * n_chunks
        out_cp = [None] * n_chunks
        in_cp[0] = pltpu.async_copy(
            idx_hbm.at[pl.ds(row0, chunk), :], idx_bufs[0], in_sems[0])
        pltpu.sync_copy(lut_hbm, lut_v)
        for c in range(n_chunks):
            crow = row0 + c * chunk
            if c + 1 < n_chunks:
                in_cp[c + 1] = pltpu.async_copy(
                    idx_hbm.at[pl.ds(crow + chunk, chunk), :],
                    idx_bufs[(c + 1) % 2], in_sems[(c + 1) % 2])
            in_cp[c].wait()
            if c >= 2:
                out_cp[c - 2].wait()
            compute(idx_bufs[c % 2], out_bufs[c % 2])
            out_cp[c] = pltpu.async_copy(
                out_bufs[c % 2], out_hbm.at[pl.ds(crow, chunk), :],
                out_sems[c % 2])
        out_cp[n_chunks - 2].wait()
        out_cp[n_chunks - 1].wait()

    return sc_lookup(idx2d, lut_flat)


# speedup vs baseline: 1.7854x; 1.6004x over previous
import functools

import jax
import jax.numpy as jnp
from jax import lax
from jax.experimental import pallas as pl
from jax.experimental.pallas import tpu as pltpu
from jax.experimental.pallas import tpu_sc as plsc

_NUM_CORES = 2
_NUM_SUBCORES = 16
_LANES = 16
_NW = _NUM_CORES * _NUM_SUBCORES


def _lut_body(vocab, k_per_row, emb_ref, w_ref, b_ref, lut_ref):
    n_combo = vocab ** k_per_row
    r = lax.broadcasted_iota(jnp.int32, (n_combo, vocab), 0)
    v = lax.broadcasted_iota(jnp.int32, (n_combo, vocab), 1)
    counts = jnp.zeros((n_combo, vocab), jnp.float32)
    for slot in range(k_per_row):
        digit = (r // (vocab ** (k_per_row - 1 - slot))) % vocab
        counts = counts + (digit == v).astype(jnp.float32)
    counts = counts * (1.0 / k_per_row)
    pooled = jnp.dot(counts, emb_ref[...], preferred_element_type=jnp.float32)
    logits = jnp.dot(pooled, w_ref[...], preferred_element_type=jnp.float32)
    logits = logits + b_ref[...]
    m = jnp.max(logits, axis=-1, keepdims=True)
    e = jnp.exp(logits - m)
    lut_ref[...] = e / jnp.sum(e, axis=-1, keepdims=True)


def kernel(inputs, emb_table, W, b):
    batch, k_per_row = inputs.shape
    vocab = emb_table.shape[0]
    out_units = W.shape[1]
    n_combo = vocab ** k_per_row
    n_lut = n_combo * out_units

    lut = pl.pallas_call(
        functools.partial(_lut_body, vocab, k_per_row),
        out_shape=jax.ShapeDtypeStruct((n_combo, out_units), jnp.float32),
    )(emb_table, W, b.reshape(1, out_units)).reshape(n_lut)

    idxT = inputs.astype(jnp.int32).T.reshape(-1)
    rows_per_w = batch // _NW
    iters = rows_per_w // _LANES

    mesh = plsc.VectorSubcoreMesh(
        core_axis_name="c", subcore_axis_name="s",
        num_cores=_NUM_CORES, num_subcores=_NUM_SUBCORES)

    @functools.partial(
        pl.kernel,
        out_type=jax.ShapeDtypeStruct((out_units * batch,), jnp.float32),
        mesh=mesh,
        compiler_params=pltpu.CompilerParams(needs_layout_passes=False),
        scratch_types=[
            pltpu.VMEM((k_per_row * rows_per_w,), jnp.int32),
            pltpu.VMEM((n_lut,), jnp.float32),
            pltpu.VMEM((out_units * rows_per_w,), jnp.float32),
            pltpu.SemaphoreType.DMA,
        ],
    )
    def sc_lookup(idxT_hbm, lut_hbm, outT_hbm, idx_v, lut_v, out_v, sem):
        wid = lax.axis_index("s") * _NUM_CORES + lax.axis_index("c")
        row0 = wid * rows_per_w
        in_cps = [
            pltpu.async_copy(
                idxT_hbm.at[pl.ds(s * batch + row0, rows_per_w)],
                idx_v.at[pl.ds(s * rows_per_w, rows_per_w)], sem)
            for s in range(k_per_row)]
        pltpu.sync_copy(lut_hbm, lut_v)
        for cp in in_cps:
            cp.wait()

        @plsc.parallel_loop(0, iters, unroll=8)
        def body(j):
            r0 = j * _LANES
            i0 = idx_v[pl.ds(r0, _LANES)]
            i1 = idx_v[pl.ds(rows_per_w + r0, _LANES)]
            i2 = idx_v[pl.ds(2 * rows_per_w + r0, _LANES)]
            c3 = (i0 * (vocab * vocab) + i1 * vocab + i2) * out_units
            for k in range(out_units):
                out_v[pl.ds(k * rows_per_w + r0, _LANES)] = (
                    plsc.load_gather(lut_v, [c3 + k]))

        for k in range(out_units):
            pltpu.sync_copy(
                out_v.at[pl.ds(k * rows_per_w, rows_per_w)],
                outT_hbm.at[pl.ds(k * batch + row0, rows_per_w)])

    out_flat = sc_lookup(idxT, lut)
    return out_flat.reshape(out_units, batch).T


# speedup vs baseline: 1.9255x; 1.0785x over previous
import functools

import jax
import jax.numpy as jnp
from jax import lax
from jax.experimental import pallas as pl
from jax.experimental.pallas import tpu as pltpu
from jax.experimental.pallas import tpu_sc as plsc

_NUM_CORES = 1
_NUM_SUBCORES = 16
_LANES = 16
_NW = _NUM_CORES * _NUM_SUBCORES


def _lut_body(vocab, k_per_row, emb_ref, w_ref, b_ref, lut_ref):
    n_combo = vocab ** k_per_row
    r = lax.broadcasted_iota(jnp.int32, (n_combo, vocab), 0)
    v = lax.broadcasted_iota(jnp.int32, (n_combo, vocab), 1)
    counts = jnp.zeros((n_combo, vocab), jnp.float32)
    for slot in range(k_per_row):
        digit = (r // (vocab ** (k_per_row - 1 - slot))) % vocab
        counts = counts + (digit == v).astype(jnp.float32)
    counts = counts * (1.0 / k_per_row)
    pooled = jnp.dot(counts, emb_ref[...], preferred_element_type=jnp.float32)
    logits = jnp.dot(pooled, w_ref[...], preferred_element_type=jnp.float32)
    logits = logits + b_ref[...]
    m = jnp.max(logits, axis=-1, keepdims=True)
    e = jnp.exp(logits - m)
    lut_ref[...] = e / jnp.sum(e, axis=-1, keepdims=True)


def kernel(inputs, emb_table, W, b):
    batch, k_per_row = inputs.shape
    vocab = emb_table.shape[0]
    out_units = W.shape[1]
    n_combo = vocab ** k_per_row
    n_lut = n_combo * out_units

    lut = pl.pallas_call(
        functools.partial(_lut_body, vocab, k_per_row),
        out_shape=jax.ShapeDtypeStruct((n_combo, out_units), jnp.float32),
    )(emb_table, W, b.reshape(1, out_units)).reshape(n_lut)

    idxT = inputs.astype(jnp.int32).T.reshape(-1)
    rows_per_w = batch // _NW
    iters = rows_per_w // _LANES

    mesh = plsc.VectorSubcoreMesh(
        core_axis_name="c", subcore_axis_name="s",
        num_cores=_NUM_CORES, num_subcores=_NUM_SUBCORES)

    @functools.partial(
        pl.kernel,
        out_type=jax.ShapeDtypeStruct((out_units * batch,), jnp.float32),
        mesh=mesh,
        compiler_params=pltpu.CompilerParams(needs_layout_passes=False),
        scratch_types=[
            pltpu.VMEM((k_per_row * rows_per_w,), jnp.int32),
            pltpu.VMEM((n_lut,), jnp.float32),
            pltpu.VMEM((out_units * rows_per_w,), jnp.float32),
            pltpu.SemaphoreType.DMA,
        ],
    )
    def sc_lookup(idxT_hbm, lut_hbm, outT_hbm, idx_v, lut_v, out_v, sem):
        wid = lax.axis_index("s") * _NUM_CORES + lax.axis_index("c")
        row0 = wid * rows_per_w
        in_cps = [
            pltpu.async_copy(
                idxT_hbm.at[pl.ds(s * batch + row0, rows_per_w)],
                idx_v.at[pl.ds(s * rows_per_w, rows_per_w)], sem)
            for s in range(k_per_row)]
        pltpu.sync_copy(lut_hbm, lut_v)
        for cp in in_cps:
            cp.wait()

        @plsc.parallel_loop(0, iters, unroll=8)
        def body(j):
            r0 = j * _LANES
            i0 = idx_v[pl.ds(r0, _LANES)]
            i1 = idx_v[pl.ds(rows_per_w + r0, _LANES)]
            i2 = idx_v[pl.ds(2 * rows_per_w + r0, _LANES)]
            c3 = (i0 * (vocab * vocab) + i1 * vocab + i2) * out_units
            for k in range(out_units):
                out_v[pl.ds(k * rows_per_w + r0, _LANES)] = (
                    plsc.load_gather(lut_v, [c3 + k]))

        for k in range(out_units):
            pltpu.sync_copy(
                out_v.at[pl.ds(k * rows_per_w, rows_per_w)],
                outT_hbm.at[pl.ds(k * batch + row0, rows_per_w)])

    out_flat = sc_lookup(idxT, lut)
    return out_flat.reshape(out_units, batch).T
